# baseline (speedup 1.0000x reference)
import functools

import jax
import jax.numpy as jnp
from jax.experimental import pallas as pl

B = 2
N = 4096
K = 10
RBLK = 2048
NSWEEP = 7


def _jacobi3(a00, a01, a02, a11, a12, a22):
    one = jnp.float32(1.0)
    zero = jnp.float32(0.0)
    A = {(0, 0): a00, (0, 1): a01, (0, 2): a02,
         (1, 1): a11, (1, 2): a12, (2, 2): a22}
    V = {}
    for i in range(3):
        for j in range(3):
            V[(i, j)] = jnp.full_like(a00, one if i == j else zero)

    def getA(i, j):
        return A[(i, j)] if i <= j else A[(j, i)]

    def setA(i, j, val):
        A[(min(i, j), max(i, j))] = val

    for _ in range(NSWEEP):
        for (p, q) in ((0, 2), (1, 2), (0, 1)):
            app = getA(p, p)
            aqq = getA(q, q)
            apq = getA(p, q)
            nz = apq != zero
            apq_safe = jnp.where(nz, apq, one)
            tau = (aqq - app) / (2.0 * apq_safe)
            rt = jnp.sqrt(one + tau * tau)
            t = jnp.where(tau >= zero, one / (tau + rt), -one / (rt - tau))
            c = jax.lax.rsqrt(one + t * t)
            s = t * c
            c = jnp.where(nz, c, one)
            s = jnp.where(nz, s, zero)
            r = 3 - p - q
            apr = getA(p, r)
            aqr = getA(q, r)
            setA(p, p, c * c * app - 2.0 * s * c * apq + s * s * aqq)
            setA(q, q, s * s * app + 2.0 * s * c * apq + c * c * aqq)
            setA(p, q, s * c * (app - aqq) + (c * c - s * s) * apq)
            setA(p, r, c * apr - s * aqr)
            setA(q, r, s * apr + c * aqr)
            for i in range(3):
                vip = V[(i, p)]
                viq = V[(i, q)]
                V[(i, p)] = c * vip - s * viq
                V[(i, q)] = s * vip + c * viq
    return V, (A[(0, 0)], A[(1, 1)], A[(2, 2)])


def _normals_ptof(sx, sy, sz, sxx, sxy, sxz, syy, syz, szz, px, py, pz):
    invk = jnp.float32(1.0 / K)
    cx = sx * invk
    cy = sy * invk
    cz = sz * invk
    g00 = sxx - sx * sx * invk
    g01 = sxy - sx * sy * invk
    g02 = sxz - sx * sz * invk
    g11 = syy - sy * sy * invk
    g12 = syz - sy * sz * invk
    g22 = szz - sz * sz * invk

    W, (l0, l1, l2) = _jacobi3(g00, g01, g02, g11, g12, g22)
    r0 = jnp.sqrt(jnp.maximum(l0, 0.0))
    r1 = jnp.sqrt(jnp.maximum(l1, 0.0))
    r2 = jnp.sqrt(jnp.maximum(l2, 0.0))
    h = {}
    for i in range(3):
        for j in range(i, 3):
            h[(i, j)] = (W[(i, 0)] * r0 * W[(j, 0)]
                         + W[(i, 1)] * r1 * W[(j, 1)]
                         + W[(i, 2)] * r2 * W[(j, 2)])
    V, (w0, w1, w2) = _jacobi3(h[(0, 0)], h[(0, 1)], h[(0, 2)],
                               h[(1, 1)], h[(1, 2)], h[(2, 2)])
    w0 = jnp.maximum(w0, 0.0)
    w1 = jnp.maximum(w1, 0.0)
    w2 = jnp.maximum(w2, 0.0)
    m = jnp.minimum(jnp.minimum(w0, w1), w2)
    sel2 = w2 == m
    sel1 = w1 == m
    nx = jnp.where(sel2, V[(0, 2)], jnp.where(sel1, V[(0, 1)], V[(0, 0)]))
    ny = jnp.where(sel2, V[(1, 2)], jnp.where(sel1, V[(1, 1)], V[(1, 0)]))
    nz = jnp.where(sel2, V[(2, 2)], jnp.where(sel1, V[(2, 1)], V[(2, 0)]))

    ptof = (px - cx) * nx + (py - cy) * ny + (pz - cz) * nz
    return ptof


def _body(x1b_ref, x1t_ref, x1tb_ref, x2t_ref, x2tb_ref, out_ref):
    i0 = pl.program_id(0)
    i1 = pl.program_id(1)

    x1b = x1b_ref[0]
    x1t = x1t_ref[0]
    x1tb = x1tb_ref[0]
    x2t = x2t_ref[0]
    x2tb = x2tb_ref[0]

    d2_all = jnp.sum(x1t * x1t, axis=0, keepdims=True)
    d2_blk = jnp.sum(x1b * x1b, axis=1, keepdims=True)
    dot = jax.lax.dot_general(
        x1b.astype(jnp.bfloat16), x1t.astype(jnp.bfloat16),
        (((1,), (0,)), ((), ())),
        preferred_element_type=jnp.float32)
    dist = d2_blk + d2_all - 2.0 * dot

    inf = jnp.float32(jnp.inf)
    mval = jnp.min(dist, axis=1, keepdims=True)
    for _ in range(K - 1):
        mval = jnp.min(jnp.where(dist > mval, dist, inf),
                       axis=1, keepdims=True)

    f1 = [x1t[0:1], x1t[1:2], x1t[2:3],
          x1t[0:1] * x1t[0:1], x1t[0:1] * x1t[1:2], x1t[0:1] * x1t[2:3],
          x1t[1:2] * x1t[1:2], x1t[1:2] * x1t[2:3], x1t[2:3] * x1t[2:3]]
    f2 = [x2t[0:1], x2t[1:2], x2t[2:3],
          x2t[0:1] * x2t[0:1], x2t[0:1] * x2t[1:2], x2t[0:1] * x2t[2:3],
          x2t[1:2] * x2t[1:2], x2t[1:2] * x2t[2:3], x2t[2:3] * x2t[2:3]]
    feats = jnp.concatenate(f1 + f2, axis=0)
    mb = (dist <= mval).astype(jnp.bfloat16)
    f_hi = feats.astype(jnp.bfloat16)
    rem = feats - f_hi.astype(jnp.float32)
    f_mid = rem.astype(jnp.bfloat16)
    f_lo = (rem - f_mid.astype(jnp.float32)).astype(jnp.bfloat16)
    dims = (((1,), (1,)), ((), ()))

    def mm(a):
        return jax.lax.dot_general(a, mb, dims,
                                   preferred_element_type=jnp.float32)

    S = mm(f_hi) + mm(f_mid) + mm(f_lo)

    def rs(a):
        return a.reshape(RBLK // 128, 128)

    ptof1 = _normals_ptof(rs(S[0:1]), rs(S[1:2]), rs(S[2:3]), rs(S[3:4]),
                          rs(S[4:5]), rs(S[5:6]), rs(S[6:7]), rs(S[7:8]),
                          rs(S[8:9]),
                          rs(x1tb[0:1]), rs(x1tb[1:2]), rs(x1tb[2:3]))
    ptof2 = _normals_ptof(rs(S[9:10]), rs(S[10:11]), rs(S[11:12]),
                          rs(S[12:13]), rs(S[13:14]), rs(S[14:15]),
                          rs(S[15:16]), rs(S[16:17]), rs(S[17:18]),
                          rs(x2tb[0:1]), rs(x2tb[1:2]), rs(x2tb[2:3]))

    d_abs = jnp.abs(ptof1) - jnp.abs(ptof2)
    bent = jnp.maximum(ptof2 - ptof1, 0.0)
    partial = jnp.sum(d_abs * d_abs + 5.0 * bent * bent,
                      keepdims=True).reshape(1, 1)

    first = jnp.logical_and(i0 == 0, i1 == 0)

    @pl.when(first)
    def _():
        out_ref[...] = partial

    @pl.when(jnp.logical_not(first))
    def _():
        out_ref[...] = out_ref[...] + partial


@jax.jit
def kernel(xyz1, xyz2):
    x1t = jnp.swapaxes(xyz1, 1, 2)
    x2t = jnp.swapaxes(xyz2, 1, 2)

    nblk = N // RBLK
    grid = (B, nblk)
    res = pl.pallas_call(
        _body,
        grid=grid,
        in_specs=[
            pl.BlockSpec((1, RBLK, 3), lambda b, r: (b, r, 0)),
            pl.BlockSpec((1, 3, N), lambda b, r: (b, 0, 0)),
            pl.BlockSpec((1, 3, RBLK), lambda b, r: (b, 0, r)),
            pl.BlockSpec((1, 3, N), lambda b, r: (b, 0, 0)),
            pl.BlockSpec((1, 3, RBLK), lambda b, r: (b, 0, r)),
        ],
        out_specs=pl.BlockSpec((1, 1), lambda b, r: (0, 0)),
        out_shape=jax.ShapeDtypeStruct((1, 1), jnp.float32),
    )(xyz1, x1t, x1t, x2t, x2t)
    return res[0, 0] * jnp.float32(1.0 / (B * N))


# speedup vs baseline: 1.1927x; 1.1927x over previous
import functools

import jax
import jax.numpy as jnp
from jax.experimental import pallas as pl

B = 2
N = 4096
K = 10
RBLK = 1024
NSWEEP = 7


def _jacobi3(a00, a01, a02, a11, a12, a22):
    one = jnp.float32(1.0)
    zero = jnp.float32(0.0)
    A = {(0, 0): a00, (0, 1): a01, (0, 2): a02,
         (1, 1): a11, (1, 2): a12, (2, 2): a22}
    V = {}
    for i in range(3):
        for j in range(3):
            V[(i, j)] = jnp.full_like(a00, one if i == j else zero)

    def getA(i, j):
        return A[(i, j)] if i <= j else A[(j, i)]

    def setA(i, j, val):
        A[(min(i, j), max(i, j))] = val

    for _ in range(NSWEEP):
        for (p, q) in ((0, 2), (1, 2), (0, 1)):
            app = getA(p, p)
            aqq = getA(q, q)
            apq = getA(p, q)
            nz = apq != zero
            apq_safe = jnp.where(nz, apq, one)
            tau = (aqq - app) / (2.0 * apq_safe)
            rt = jnp.sqrt(one + tau * tau)
            t = jnp.where(tau >= zero, one / (tau + rt), -one / (rt - tau))
            c = jax.lax.rsqrt(one + t * t)
            s = t * c
            c = jnp.where(nz, c, one)
            s = jnp.where(nz, s, zero)
            r = 3 - p - q
            apr = getA(p, r)
            aqr = getA(q, r)
            setA(p, p, c * c * app - 2.0 * s * c * apq + s * s * aqq)
            setA(q, q, s * s * app + 2.0 * s * c * apq + c * c * aqq)
            setA(p, q, s * c * (app - aqq) + (c * c - s * s) * apq)
            setA(p, r, c * apr - s * aqr)
            setA(q, r, s * apr + c * aqr)
            for i in range(3):
                vip = V[(i, p)]
                viq = V[(i, q)]
                V[(i, p)] = c * vip - s * viq
                V[(i, q)] = s * vip + c * viq
    return V, (A[(0, 0)], A[(1, 1)], A[(2, 2)])


def _normals_ptof(sx, sy, sz, sxx, sxy, sxz, syy, syz, szz, px, py, pz):
    invk = jnp.float32(1.0 / K)
    cx = sx * invk
    cy = sy * invk
    cz = sz * invk
    g00 = sxx - sx * sx * invk
    g01 = sxy - sx * sy * invk
    g02 = sxz - sx * sz * invk
    g11 = syy - sy * sy * invk
    g12 = syz - sy * sz * invk
    g22 = szz - sz * sz * invk

    W, (l0, l1, l2) = _jacobi3(g00, g01, g02, g11, g12, g22)
    r0 = jnp.sqrt(jnp.maximum(l0, 0.0))
    r1 = jnp.sqrt(jnp.maximum(l1, 0.0))
    r2 = jnp.sqrt(jnp.maximum(l2, 0.0))
    h = {}
    for i in range(3):
        for j in range(i, 3):
            h[(i, j)] = (W[(i, 0)] * r0 * W[(j, 0)]
                         + W[(i, 1)] * r1 * W[(j, 1)]
                         + W[(i, 2)] * r2 * W[(j, 2)])
    V, (w0, w1, w2) = _jacobi3(h[(0, 0)], h[(0, 1)], h[(0, 2)],
                               h[(1, 1)], h[(1, 2)], h[(2, 2)])
    w0 = jnp.maximum(w0, 0.0)
    w1 = jnp.maximum(w1, 0.0)
    w2 = jnp.maximum(w2, 0.0)
    m = jnp.minimum(jnp.minimum(w0, w1), w2)
    sel2 = w2 == m
    sel1 = w1 == m
    nx = jnp.where(sel2, V[(0, 2)], jnp.where(sel1, V[(0, 1)], V[(0, 0)]))
    ny = jnp.where(sel2, V[(1, 2)], jnp.where(sel1, V[(1, 1)], V[(1, 0)]))
    nz = jnp.where(sel2, V[(2, 2)], jnp.where(sel1, V[(2, 1)], V[(2, 0)]))

    ptof = (px - cx) * nx + (py - cy) * ny + (pz - cz) * nz
    return ptof


def _body(x1b_ref, x1t_ref, x1tb_ref, x2t_ref, x2tb_ref, out_ref):
    i0 = pl.program_id(0)
    i1 = pl.program_id(1)

    x1b = x1b_ref[0]
    x1t = x1t_ref[0]
    x1tb = x1tb_ref[0]
    x2t = x2t_ref[0]
    x2tb = x2tb_ref[0]

    d2_all = jnp.sum(x1t * x1t, axis=0, keepdims=True)
    d2_blk = jnp.sum(x1b * x1b, axis=1, keepdims=True)
    dot = jax.lax.dot_general(
        x1b.astype(jnp.bfloat16), x1t.astype(jnp.bfloat16),
        (((1,), (0,)), ((), ())),
        preferred_element_type=jnp.float32)
    dist = d2_blk + d2_all - 2.0 * dot

    inf = jnp.float32(jnp.inf)
    mval = jnp.min(dist, axis=1, keepdims=True)
    for _ in range(K - 1):
        mval = jnp.min(jnp.where(dist > mval, dist, inf),
                       axis=1, keepdims=True)

    f1 = [x1t[0:1], x1t[1:2], x1t[2:3],
          x1t[0:1] * x1t[0:1], x1t[0:1] * x1t[1:2], x1t[0:1] * x1t[2:3],
          x1t[1:2] * x1t[1:2], x1t[1:2] * x1t[2:3], x1t[2:3] * x1t[2:3]]
    f2 = [x2t[0:1], x2t[1:2], x2t[2:3],
          x2t[0:1] * x2t[0:1], x2t[0:1] * x2t[1:2], x2t[0:1] * x2t[2:3],
          x2t[1:2] * x2t[1:2], x2t[1:2] * x2t[2:3], x2t[2:3] * x2t[2:3]]
    feats = jnp.concatenate(f1 + f2, axis=0)
    mb = (dist <= mval).astype(jnp.bfloat16)
    f_hi = feats.astype(jnp.bfloat16)
    rem = feats - f_hi.astype(jnp.float32)
    f_mid = rem.astype(jnp.bfloat16)
    f_lo = (rem - f_mid.astype(jnp.float32)).astype(jnp.bfloat16)
    dims = (((1,), (1,)), ((), ()))

    def mm(a):
        return jax.lax.dot_general(a, mb, dims,
                                   preferred_element_type=jnp.float32)

    S = mm(f_hi) + mm(f_mid) + mm(f_lo)

    def rs(a):
        return a.reshape(RBLK // 128, 128)

    ptof1 = _normals_ptof(rs(S[0:1]), rs(S[1:2]), rs(S[2:3]), rs(S[3:4]),
                          rs(S[4:5]), rs(S[5:6]), rs(S[6:7]), rs(S[7:8]),
                          rs(S[8:9]),
                          rs(x1tb[0:1]), rs(x1tb[1:2]), rs(x1tb[2:3]))
    ptof2 = _normals_ptof(rs(S[9:10]), rs(S[10:11]), rs(S[11:12]),
                          rs(S[12:13]), rs(S[13:14]), rs(S[14:15]),
                          rs(S[15:16]), rs(S[16:17]), rs(S[17:18]),
                          rs(x2tb[0:1]), rs(x2tb[1:2]), rs(x2tb[2:3]))

    d_abs = jnp.abs(ptof1) - jnp.abs(ptof2)
    bent = jnp.maximum(ptof2 - ptof1, 0.0)
    partial = jnp.sum(d_abs * d_abs + 5.0 * bent * bent,
                      keepdims=True).reshape(1, 1)

    first = jnp.logical_and(i0 == 0, i1 == 0)

    @pl.when(first)
    def _():
        out_ref[...] = partial

    @pl.when(jnp.logical_not(first))
    def _():
        out_ref[...] = out_ref[...] + partial


@jax.jit
def kernel(xyz1, xyz2):
    x1t = jnp.swapaxes(xyz1, 1, 2)
    x2t = jnp.swapaxes(xyz2, 1, 2)

    nblk = N // RBLK
    grid = (B, nblk)
    res = pl.pallas_call(
        _body,
        grid=grid,
        in_specs=[
            pl.BlockSpec((1, RBLK, 3), lambda b, r: (b, r, 0)),
            pl.BlockSpec((1, 3, N), lambda b, r: (b, 0, 0)),
            pl.BlockSpec((1, 3, RBLK), lambda b, r: (b, 0, r)),
            pl.BlockSpec((1, 3, N), lambda b, r: (b, 0, 0)),
            pl.BlockSpec((1, 3, RBLK), lambda b, r: (b, 0, r)),
        ],
        out_specs=pl.BlockSpec((1, 1), lambda b, r: (0, 0)),
        out_shape=jax.ShapeDtypeStruct((1, 1), jnp.float32),
    )(xyz1, x1t, x1t, x2t, x2t)
    return res[0, 0] * jnp.float32(1.0 / (B * N))
